# Initial kernel scaffold; baseline (speedup 1.0000x reference)
import jax
import jax.numpy as jnp
from jax.experimental import pallas as pl


def kernel(idx, table):
    raise NotImplementedError("write your pallas kernel here")


# speedup vs baseline: 14.6183x; 14.6183x over previous
import functools

import jax
import jax.numpy as jnp
from jax import lax
from jax.experimental import pallas as pl
from jax.experimental.pallas import tpu as pltpu
from jax.experimental.pallas import tpu_sc as plsc

MAX_NORM = 0.175

NUM_ROWS = 150000
NPAD = 150528
NCOL = 1176
B = 16384 * 200
IDX_ROWS = B // 128
NW = 32
ROWS_PER_W = IDX_ROWS // NW
K = 8
D = 8


def _prep_body(x_ref, y_ref, z_ref, o_ref):
    x = x_ref[...]
    y = y_ref[...]
    z = z_ref[...]
    n = jnp.sqrt(x * x + y * y + z * z)
    scale = jnp.where(n > MAX_NORM, MAX_NORM / jnp.maximum(n, 1e-7), 1.0)
    xs = x * scale
    ys = y * scale
    zs = z * scale
    o_ref[0] = -zs
    o_ref[1] = ys
    o_ref[2] = zs
    o_ref[3] = -xs
    o_ref[4] = -ys
    o_ref[5] = xs
    o_ref[6] = jnp.zeros_like(xs)
    o_ref[7] = jnp.zeros_like(xs)


_prep = pl.pallas_call(
    _prep_body,
    out_shape=jax.ShapeDtypeStruct((D, NCOL, 128), jnp.float32),
)


def _gather_body(t6_hbm, idx_hbm, out_hbm, idx_v, rows_v, sem):
    c = lax.axis_index("c")
    s = lax.axis_index("s")
    wid = s * 2 + c
    base = wid * ROWS_PER_W

    def chunk(i, carry):
        rb = base + i * K
        pltpu.sync_copy(idx_hbm.at[pl.ds(rb, K)], idx_v)
        handles = [
            pltpu.async_copy(t6_hbm.at[idx_v.at[j]], rows_v.at[j], sem)
            for j in range(K)
        ]
        for h in handles:
            h.wait()
        pltpu.sync_copy(rows_v, out_hbm.at[pl.ds(rb, K)])
        return carry

    lax.fori_loop(0, ROWS_PER_W // K, chunk, 0)


@functools.cache
def _make_gather():
    return pl.kernel(
        _gather_body,
        mesh=plsc.VectorSubcoreMesh(core_axis_name="c", subcore_axis_name="s"),
        compiler_params=pltpu.CompilerParams(use_tc_tiling_on_sc=False),
        out_type=jax.ShapeDtypeStruct((IDX_ROWS, 128, D), jnp.float32),
        scratch_types=[
            pltpu.VMEM((K, 128), jnp.int32),
            pltpu.VMEM((K, 128, D), jnp.float32),
            pltpu.SemaphoreType.DMA,
        ],
    )


def kernel(idx, table):
    nb, nl = idx.shape
    table_p = jnp.zeros((NPAD, 3), jnp.float32).at[:NUM_ROWS].set(table)
    xc = table_p[:, 0].reshape(NCOL, 128)
    yc = table_p[:, 1].reshape(NCOL, 128)
    zc = table_p[:, 2].reshape(NCOL, 128)
    cols = _prep(xc, yc, zc)
    t6 = jnp.transpose(cols, (1, 2, 0)).reshape(NPAD, D)
    idx2d = idx.astype(jnp.int32).reshape(IDX_ROWS, 128)
    out = _make_gather()(t6, idx2d)
    return out.reshape(B, D)[:, :6].reshape(nb, nl, 3, 2)
